# baseline (speedup 1.0000x reference)
import functools

import jax
import jax.numpy as jnp
from jax import lax
from jax.experimental import pallas as pl
from jax.experimental.pallas import tpu as pltpu
from jax.experimental.pallas import tpu_sc as plsc

N_NODES = 10000
NPAD = 10240
HALF_N = NPAD // 2
D = 128
E = 320000
NC = 2
NS = 16
K = 128
ITERS = 160
EPS = ITERS * K
E_PAD = EPS * NS
DEPTH = 4
LOOK = 2
AROWS = HALF_N + 128
RPS = AROWS // NS
SHIFT = 14
DUMP_RAW = 3 * HALF_N

_sc_mesh = plsc.VectorSubcoreMesh(core_axis_name="c", subcore_axis_name="s")


@functools.partial(
    pl.kernel,
    mesh=_sc_mesh,
    out_type=jax.ShapeDtypeStruct((NC, AROWS, D), jnp.float32),
    scratch_types=[
        pltpu.VMEM((ITERS, K), jnp.int32),
        pltpu.VMEM((LOOK, K), jnp.int32),
        pltpu.VMEM((DEPTH, K), jnp.int32),
        pltpu.VMEM((K, D), jnp.float32),
        pltpu.VMEM((K, D), jnp.float32),
        pltpu.VMEM((K, D), jnp.float32),
        pltpu.VMEM((K, D), jnp.float32),
        pltpu.VMEM_SHARED((AROWS, D), jnp.float32),
        pltpu.SemaphoreType.DMA((DEPTH,)),
        pltpu.SemaphoreType.DMA((DEPTH,)),
    ],
)
def _sc_agg(hs_hbm, pk_hbm, zeros_hbm, out_hbm,
            pk_v, srcb, dstb, rows_a, rows_b, rows_c, rows_d,
            agg_sh, gsem, ssem):
    rows = (rows_a, rows_b, rows_c, rows_d)
    cid = lax.axis_index("c")
    sid = lax.axis_index("s")
    half = cid * HALF_N

    pltpu.sync_copy(zeros_hbm.at[pl.ds(sid * RPS, RPS)],
                    agg_sh.at[pl.ds(sid * RPS, RPS)])
    pltpu.sync_copy(pk_hbm.at[sid], pk_v)
    plsc.subcore_barrier()

    def unpack(c, sb, db):
        for j in range(K // 16):
            v = pk_v[c, pl.ds(j * 16, 16)]
            srcb[sb, pl.ds(j * 16, 16)] = v & ((1 << SHIFT) - 1)
            dg = lax.shift_right_logical(v, SHIFT) - half
            ok = (dg >= 0) & (dg < HALF_N)
            dstb[db, pl.ds(j * 16, 16)] = jnp.where(ok, dg, HALF_N)

    def gather(b, sb):
        pltpu.async_copy(hs_hbm.at[srcb.at[sb]], rows[b], gsem.at[b])

    def gwait(b):
        pltpu.make_async_copy(hs_hbm.at[pl.ds(0, K)], rows[b],
                              gsem.at[b]).wait()

    def scat(b):
        pltpu.async_copy(rows[b], agg_sh.at[dstb.at[b]], ssem.at[b],
                         add=True)

    def swait(b):
        pltpu.make_async_copy(hs_hbm.at[pl.ds(0, K)], rows[b],
                              ssem.at[b]).wait()

    for c in range(LOOK):
        unpack(c, c % LOOK, c % DEPTH)
        gather(c % DEPTH, c % LOOK)
    for c in range(DEPTH - LOOK):
        gwait(c % DEPTH)
        scat(c % DEPTH)
        unpack(c + LOOK, c % LOOK, (c + LOOK) % DEPTH)
        gather((c + LOOK) % DEPTH, c % LOOK)

    def outer(go, carry):
        base = go * DEPTH + (DEPTH - LOOK)
        for j in range(DEPTH):
            b = (DEPTH - LOOK + j) % DEPTH
            bg = j % DEPTH
            gwait(b)
            scat(b)
            swait(bg)
            unpack(base + j + LOOK, j % LOOK, bg)
            gather(bg, j % LOOK)
        return carry

    lax.fori_loop(0, (ITERS - DEPTH) // DEPTH, outer, 0)

    for j in range(LOOK):
        b = (ITERS - LOOK + j) % DEPTH
        gwait(b)
        scat(b)
    for b in range(DEPTH):
        swait(b)

    plsc.subcore_barrier()
    pltpu.sync_copy(agg_sh.at[pl.ds(sid * RPS, RPS)],
                    out_hbm.at[cid, pl.ds(sid * RPS, RPS)])


_BR = 1024
_NBLK = HALF_N // _BR


def _agg_spec():
    return pl.BlockSpec((1, _BR, D), lambda i: (i // _NBLK, i % _NBLK, 0))


def _tc_first_body(x_ref, w_ref, degp_ref, hs_ref, dinv_ref):
    deg = degp_ref[0, :, 0:1] + 1.0
    dinv = lax.rsqrt(deg)
    h = jnp.dot(x_ref[...], w_ref[...], preferred_element_type=jnp.float32)
    hs_ref[...] = h * dinv
    dinv_ref[...] = dinv


_tc_first = pl.pallas_call(
    _tc_first_body,
    grid=(NPAD // _BR,),
    in_specs=[
        pl.BlockSpec((_BR, D), lambda i: (i, 0)),
        pl.BlockSpec((D, D), lambda i: (0, 0)),
        _agg_spec(),
    ],
    out_specs=[
        pl.BlockSpec((_BR, D), lambda i: (i, 0)),
        pl.BlockSpec((_BR, 1), lambda i: (i, 0)),
    ],
    out_shape=[
        jax.ShapeDtypeStruct((NPAD, D), jnp.float32),
        jax.ShapeDtypeStruct((NPAD, 1), jnp.float32),
    ],
)


def _tc_mid_body(aggp_ref, hs_ref, dinv_ref, b_ref, w_ref, out_ref):
    dinv = dinv_ref[...]
    pre = dinv * (aggp_ref[0] + hs_ref[...]) + b_ref[...]
    pre = jnp.maximum(pre, 0.0)
    h = jnp.dot(pre, w_ref[...], preferred_element_type=jnp.float32)
    out_ref[...] = h * dinv


def _tc_last_body(aggp_ref, hs_ref, dinv_ref, b_ref, w_ref, bout_ref, out_ref):
    dinv = dinv_ref[...]
    pre = dinv * (aggp_ref[0] + hs_ref[...]) + b_ref[...]
    pre = jnp.maximum(pre, 0.0)
    h = jnp.dot(pre, w_ref[...], preferred_element_type=jnp.float32)
    out_ref[...] = h + bout_ref[...]


def _tc_layer_call(body, n_extra):
    in_specs = [
        _agg_spec(),
        pl.BlockSpec((_BR, D), lambda i: (i, 0)),
        pl.BlockSpec((_BR, 1), lambda i: (i, 0)),
        pl.BlockSpec((1, D), lambda i: (0, 0)),
        pl.BlockSpec((D, D), lambda i: (0, 0)),
    ]
    in_specs += [pl.BlockSpec((1, D), lambda i: (0, 0))] * n_extra
    return pl.pallas_call(
        body,
        grid=(NPAD // _BR,),
        in_specs=in_specs,
        out_specs=pl.BlockSpec((_BR, D), lambda i: (i, 0)),
        out_shape=jax.ShapeDtypeStruct((NPAD, D), jnp.float32),
    )


_tc_mid = _tc_layer_call(_tc_mid_body, 0)
_tc_last = _tc_layer_call(_tc_last_body, 1)


def kernel(x, edge_index, W0, b0, W1, b1, W2, b2, W3, b3, Wout, bout):
    src = edge_index[0].astype(jnp.int32)
    dst = edge_index[1].astype(jnp.int32)
    npad_e = E_PAD - E
    src = jnp.concatenate([src, jnp.zeros((npad_e,), jnp.int32)])
    dst = jnp.concatenate([dst, jnp.full((npad_e,), DUMP_RAW, jnp.int32)])
    packed = (src | (dst << SHIFT)).reshape(NS, ITERS, K)

    x_pad = jnp.pad(x, ((0, NPAD - N_NODES), (0, 0)))
    zeros_tab = jnp.zeros((AROWS, D), jnp.float32)
    ones_tab = jnp.ones((NPAD, D), jnp.float32)
    wout_pad = jnp.pad(Wout, ((0, 0), (0, D - Wout.shape[1])))
    bout_tab = jnp.pad(bout.reshape(1, 1), ((0, 0), (0, D - 1)))

    degp = _sc_agg(ones_tab, packed, zeros_tab)
    hs, dinv = _tc_first(x_pad, W0, degp)

    for b_prev, W in ((b0, W1), (b1, W2), (b2, W3)):
        aggp = _sc_agg(hs, packed, zeros_tab)
        hs = _tc_mid(aggp, hs, dinv, b_prev.reshape(1, D), W)

    aggp = _sc_agg(hs, packed, zeros_tab)
    out = _tc_last(aggp, hs, dinv, b3.reshape(1, D), wout_pad, bout_tab)
    return out[:N_NODES, :1]


# speedup vs baseline: 1.0512x; 1.0512x over previous
import functools

import jax
import jax.numpy as jnp
from jax import lax
from jax.experimental import pallas as pl
from jax.experimental.pallas import tpu as pltpu
from jax.experimental.pallas import tpu_sc as plsc

N_NODES = 10000
NPAD = 10240
HALF_N = NPAD // 2
D = 128
E = 320000
NC = 2
NS = 16
K = 128
ITERS = 160
EPS = ITERS * K
E_PAD = EPS * NS
DEPTH = 4
LOOK = 2
AROWS = HALF_N + 128
RPS = AROWS // NS
SHIFT = 14
DUMP_RAW = 3 * HALF_N

_sc_mesh = plsc.VectorSubcoreMesh(core_axis_name="c", subcore_axis_name="s")


@functools.partial(
    pl.kernel,
    mesh=_sc_mesh,
    out_type=jax.ShapeDtypeStruct((NC, AROWS, D), jnp.float32),
    scratch_types=[
        pltpu.VMEM((ITERS, K), jnp.int32),
        pltpu.VMEM((LOOK, K), jnp.int32),
        pltpu.VMEM((DEPTH, K), jnp.int32),
        pltpu.VMEM((K, D), jnp.float32),
        pltpu.VMEM((K, D), jnp.float32),
        pltpu.VMEM((K, D), jnp.float32),
        pltpu.VMEM((K, D), jnp.float32),
        pltpu.VMEM_SHARED((AROWS, D), jnp.float32),
        pltpu.SemaphoreType.DMA((DEPTH,)),
        pltpu.SemaphoreType.DMA((DEPTH,)),
    ],
)
def _sc_agg(hs_hbm, pk_hbm, zeros_hbm, out_hbm,
            pk_v, srcb, dstb, rows_a, rows_b, rows_c, rows_d,
            agg_sh, gsem, ssem):
    rows = (rows_a, rows_b, rows_c, rows_d)
    cid = lax.axis_index("c")
    sid = lax.axis_index("s")
    half = cid * HALF_N

    pltpu.sync_copy(zeros_hbm.at[pl.ds(sid * RPS, RPS)],
                    agg_sh.at[pl.ds(sid * RPS, RPS)])
    pltpu.sync_copy(pk_hbm.at[sid], pk_v)
    plsc.subcore_barrier()

    def unpack(c, sb, db):
        for j in range(K // 16):
            v = pk_v[c, pl.ds(j * 16, 16)]
            s = v & ((1 << SHIFT) - 1)
            srcb[sb, pl.ds(j * 16, 16)] = s
            dg = lax.shift_right_logical(v, SHIFT) - half
            ok = (dg >= 0) & (dg < HALF_N)
            dump = HALF_N + (s & 127)
            dstb[db, pl.ds(j * 16, 16)] = jnp.where(ok, dg, dump)

    def gather(b, sb):
        pltpu.async_copy(hs_hbm.at[srcb.at[sb]], rows[b], gsem.at[b])

    def gwait(b):
        pltpu.make_async_copy(hs_hbm.at[pl.ds(0, K)], rows[b],
                              gsem.at[b]).wait()

    def scat(b):
        pltpu.async_copy(rows[b], agg_sh.at[dstb.at[b]], ssem.at[b],
                         add=True)

    def swait(b):
        pltpu.make_async_copy(hs_hbm.at[pl.ds(0, K)], rows[b],
                              ssem.at[b]).wait()

    for c in range(LOOK):
        unpack(c, c % LOOK, c % DEPTH)
        gather(c % DEPTH, c % LOOK)
    for c in range(DEPTH - LOOK):
        gwait(c % DEPTH)
        scat(c % DEPTH)
        unpack(c + LOOK, c % LOOK, (c + LOOK) % DEPTH)
        gather((c + LOOK) % DEPTH, c % LOOK)

    def outer(go, carry):
        base = go * DEPTH + (DEPTH - LOOK)
        for j in range(DEPTH):
            b = (DEPTH - LOOK + j) % DEPTH
            bg = j % DEPTH
            gwait(b)
            scat(b)
            swait(bg)
            unpack(base + j + LOOK, j % LOOK, bg)
            gather(bg, j % LOOK)
        return carry

    lax.fori_loop(0, (ITERS - DEPTH) // DEPTH, outer, 0)

    for j in range(LOOK):
        b = (ITERS - LOOK + j) % DEPTH
        gwait(b)
        scat(b)
    for b in range(DEPTH):
        swait(b)

    plsc.subcore_barrier()
    pltpu.sync_copy(agg_sh.at[pl.ds(sid * RPS, RPS)],
                    out_hbm.at[cid, pl.ds(sid * RPS, RPS)])


_BR = 1024
_NBLK = HALF_N // _BR


def _agg_spec():
    return pl.BlockSpec((1, _BR, D), lambda i: (i // _NBLK, i % _NBLK, 0))


def _tc_first_body(x_ref, w_ref, degp_ref, hs_ref, dinv_ref):
    deg = degp_ref[0, :, 0:1] + 1.0
    dinv = lax.rsqrt(deg)
    h = jnp.dot(x_ref[...], w_ref[...], preferred_element_type=jnp.float32)
    hs_ref[...] = h * dinv
    dinv_ref[...] = dinv


_tc_first = pl.pallas_call(
    _tc_first_body,
    grid=(NPAD // _BR,),
    in_specs=[
        pl.BlockSpec((_BR, D), lambda i: (i, 0)),
        pl.BlockSpec((D, D), lambda i: (0, 0)),
        _agg_spec(),
    ],
    out_specs=[
        pl.BlockSpec((_BR, D), lambda i: (i, 0)),
        pl.BlockSpec((_BR, 1), lambda i: (i, 0)),
    ],
    out_shape=[
        jax.ShapeDtypeStruct((NPAD, D), jnp.float32),
        jax.ShapeDtypeStruct((NPAD, 1), jnp.float32),
    ],
)


def _tc_mid_body(aggp_ref, hs_ref, dinv_ref, b_ref, w_ref, out_ref):
    dinv = dinv_ref[...]
    pre = dinv * (aggp_ref[0] + hs_ref[...]) + b_ref[...]
    pre = jnp.maximum(pre, 0.0)
    h = jnp.dot(pre, w_ref[...], preferred_element_type=jnp.float32)
    out_ref[...] = h * dinv


def _tc_last_body(aggp_ref, hs_ref, dinv_ref, b_ref, w_ref, bout_ref, out_ref):
    dinv = dinv_ref[...]
    pre = dinv * (aggp_ref[0] + hs_ref[...]) + b_ref[...]
    pre = jnp.maximum(pre, 0.0)
    h = jnp.dot(pre, w_ref[...], preferred_element_type=jnp.float32)
    out_ref[...] = h + bout_ref[...]


def _tc_layer_call(body, n_extra):
    in_specs = [
        _agg_spec(),
        pl.BlockSpec((_BR, D), lambda i: (i, 0)),
        pl.BlockSpec((_BR, 1), lambda i: (i, 0)),
        pl.BlockSpec((1, D), lambda i: (0, 0)),
        pl.BlockSpec((D, D), lambda i: (0, 0)),
    ]
    in_specs += [pl.BlockSpec((1, D), lambda i: (0, 0))] * n_extra
    return pl.pallas_call(
        body,
        grid=(NPAD // _BR,),
        in_specs=in_specs,
        out_specs=pl.BlockSpec((_BR, D), lambda i: (i, 0)),
        out_shape=jax.ShapeDtypeStruct((NPAD, D), jnp.float32),
    )


_tc_mid = _tc_layer_call(_tc_mid_body, 0)
_tc_last = _tc_layer_call(_tc_last_body, 1)


def kernel(x, edge_index, W0, b0, W1, b1, W2, b2, W3, b3, Wout, bout):
    src = edge_index[0].astype(jnp.int32)
    dst = edge_index[1].astype(jnp.int32)
    npad_e = E_PAD - E
    src = jnp.concatenate([src, jnp.zeros((npad_e,), jnp.int32)])
    dst = jnp.concatenate([dst, jnp.full((npad_e,), DUMP_RAW, jnp.int32)])
    packed = (src | (dst << SHIFT)).reshape(NS, ITERS, K)

    x_pad = jnp.pad(x, ((0, NPAD - N_NODES), (0, 0)))
    zeros_tab = jnp.zeros((AROWS, D), jnp.float32)
    ones_tab = jnp.ones((NPAD, D), jnp.float32)
    wout_pad = jnp.pad(Wout, ((0, 0), (0, D - Wout.shape[1])))
    bout_tab = jnp.pad(bout.reshape(1, 1), ((0, 0), (0, D - 1)))

    degp = _sc_agg(ones_tab, packed, zeros_tab)
    hs, dinv = _tc_first(x_pad, W0, degp)

    for b_prev, W in ((b0, W1), (b1, W2), (b2, W3)):
        aggp = _sc_agg(hs, packed, zeros_tab)
        hs = _tc_mid(aggp, hs, dinv, b_prev.reshape(1, D), W)

    aggp = _sc_agg(hs, packed, zeros_tab)
    out = _tc_last(aggp, hs, dinv, b3.reshape(1, D), wout_pad, bout_tab)
    return out[:N_NODES, :1]


# speedup vs baseline: 1.6101x; 1.5317x over previous
import functools

import jax
import jax.numpy as jnp
from jax import lax
from jax.experimental import pallas as pl
from jax.experimental.pallas import tpu as pltpu
from jax.experimental.pallas import tpu_sc as plsc

N_NODES = 10000
NPAD = 10240
HALF_N = NPAD // 2
D = 128
E = 320000
NC = 2
NS = 16
NW = NC * NS
K = 128
ITERS = 80
E_PAD = NW * ITERS * K
DEPTH = 2
LOOK = 1
RPS = NPAD // NS
SHIFT = 14

_sc_mesh = plsc.VectorSubcoreMesh(core_axis_name="c", subcore_axis_name="s")


@functools.partial(
    pl.kernel,
    mesh=_sc_mesh,
    out_type=jax.ShapeDtypeStruct((NC, NPAD, D), jnp.float32),
    scratch_types=[
        pltpu.VMEM((ITERS, K), jnp.int32),
        pltpu.VMEM((LOOK, K), jnp.int32),
        pltpu.VMEM((DEPTH, K), jnp.int32),
        pltpu.VMEM((K, D), jnp.float32),
        pltpu.VMEM((K, D), jnp.float32),
        pltpu.VMEM_SHARED((NPAD, D), jnp.float32),
        pltpu.SemaphoreType.DMA((DEPTH,)),
        pltpu.SemaphoreType.DMA((DEPTH,)),
    ],
)
def _sc_agg(hs_hbm, pk_hbm, zeros_hbm, out_hbm,
            pk_v, srcb, dstb, rows_a, rows_b, agg_sh, gsem, ssem):
    rows = (rows_a, rows_b)
    cid = lax.axis_index("c")
    sid = lax.axis_index("s")
    wid = sid * NC + cid

    pltpu.sync_copy(zeros_hbm.at[pl.ds(sid * RPS, RPS)],
                    agg_sh.at[pl.ds(sid * RPS, RPS)])
    pltpu.sync_copy(pk_hbm.at[wid], pk_v)
    plsc.subcore_barrier()

    def unpack(c, db):
        for j in range(K // 16):
            v = pk_v[c, pl.ds(j * 16, 16)]
            srcb[0, pl.ds(j * 16, 16)] = v & ((1 << SHIFT) - 1)
            dstb[db, pl.ds(j * 16, 16)] = lax.shift_right_logical(v, SHIFT)

    def gather(b):
        pltpu.async_copy(hs_hbm.at[srcb.at[0]], rows[b], gsem.at[b])

    def gwait(b):
        pltpu.make_async_copy(hs_hbm.at[pl.ds(0, K)], rows[b],
                              gsem.at[b]).wait()

    def scat(b):
        pltpu.async_copy(rows[b], agg_sh.at[dstb.at[b]], ssem.at[b],
                         add=True)

    def swait(b):
        pltpu.make_async_copy(hs_hbm.at[pl.ds(0, K)], rows[b],
                              ssem.at[b]).wait()

    unpack(0, 0)
    gather(0)
    gwait(0)
    scat(0)
    unpack(1, 1)
    gather(1)

    def outer(go, carry):
        base = go * DEPTH + 1
        for j in range(DEPTH):
            b = (1 + j) % DEPTH
            bg = j % DEPTH
            gwait(b)
            scat(b)
            swait(bg)
            unpack(base + j + 1, bg)
            gather(bg)
        return carry

    lax.fori_loop(0, (ITERS - DEPTH) // DEPTH, outer, 0)

    gwait((ITERS - 1) % DEPTH)
    scat((ITERS - 1) % DEPTH)
    for b in range(DEPTH):
        swait(b)

    plsc.subcore_barrier()
    pltpu.sync_copy(agg_sh.at[pl.ds(sid * RPS, RPS)],
                    out_hbm.at[cid, pl.ds(sid * RPS, RPS)])


_BR = 1024
_NBLK = HALF_N // _BR


def _tc_first_body(x_ref, w_ref, degp_ref, hs_ref, dinv_ref):
    deg = degp_ref[0, :, 0:1] + degp_ref[1, :, 0:1] + 1.0
    dinv = lax.rsqrt(deg)
    h = jnp.dot(x_ref[...], w_ref[...], preferred_element_type=jnp.float32)
    hs_ref[...] = h * dinv
    dinv_ref[...] = dinv


_tc_first = pl.pallas_call(
    _tc_first_body,
    grid=(NPAD // _BR,),
    in_specs=[
        pl.BlockSpec((_BR, D), lambda i: (i, 0)),
        pl.BlockSpec((D, D), lambda i: (0, 0)),
        pl.BlockSpec((NC, _BR, D), lambda i: (0, i, 0)),
    ],
    out_specs=[
        pl.BlockSpec((_BR, D), lambda i: (i, 0)),
        pl.BlockSpec((_BR, 1), lambda i: (i, 0)),
    ],
    out_shape=[
        jax.ShapeDtypeStruct((NPAD, D), jnp.float32),
        jax.ShapeDtypeStruct((NPAD, 1), jnp.float32),
    ],
)


def _tc_mid_body(aggp_ref, hs_ref, dinv_ref, b_ref, w_ref, out_ref):
    dinv = dinv_ref[...]
    pre = dinv * (aggp_ref[0] + aggp_ref[1] + hs_ref[...]) + b_ref[...]
    pre = jnp.maximum(pre, 0.0)
    h = jnp.dot(pre, w_ref[...], preferred_element_type=jnp.float32)
    out_ref[...] = h * dinv


def _tc_last_body(aggp_ref, hs_ref, dinv_ref, b_ref, w_ref, bout_ref, out_ref):
    dinv = dinv_ref[...]
    pre = dinv * (aggp_ref[0] + aggp_ref[1] + hs_ref[...]) + b_ref[...]
    pre = jnp.maximum(pre, 0.0)
    h = jnp.dot(pre, w_ref[...], preferred_element_type=jnp.float32)
    out_ref[...] = h + bout_ref[...]


def _tc_layer_call(body, n_extra):
    in_specs = [
        pl.BlockSpec((NC, _BR, D), lambda i: (0, i, 0)),
        pl.BlockSpec((_BR, D), lambda i: (i, 0)),
        pl.BlockSpec((_BR, 1), lambda i: (i, 0)),
        pl.BlockSpec((1, D), lambda i: (0, 0)),
        pl.BlockSpec((D, D), lambda i: (0, 0)),
    ]
    in_specs += [pl.BlockSpec((1, D), lambda i: (0, 0))] * n_extra
    return pl.pallas_call(
        body,
        grid=(NPAD // _BR,),
        in_specs=in_specs,
        out_specs=pl.BlockSpec((_BR, D), lambda i: (i, 0)),
        out_shape=jax.ShapeDtypeStruct((NPAD, D), jnp.float32),
    )


_tc_mid = _tc_layer_call(_tc_mid_body, 0)
_tc_last = _tc_layer_call(_tc_last_body, 1)


def kernel(x, edge_index, W0, b0, W1, b1, W2, b2, W3, b3, Wout, bout):
    src = edge_index[0].astype(jnp.int32)
    dst = edge_index[1].astype(jnp.int32)
    npad_e = E_PAD - E
    src = jnp.concatenate([src, jnp.zeros((npad_e,), jnp.int32)])
    dst = jnp.concatenate(
        [dst, N_NODES + (jnp.arange(npad_e, dtype=jnp.int32) % 240)])
    packed = src | (dst << SHIFT)
    packed32 = packed.reshape(NW, ITERS, K)

    x_pad = jnp.pad(x, ((0, NPAD - N_NODES), (0, 0)))
    zeros_tab = jnp.zeros((NPAD, D), jnp.float32)
    ones_tab = jnp.ones((NPAD, D), jnp.float32)
    wout_pad = jnp.pad(Wout, ((0, 0), (0, D - Wout.shape[1])))
    bout_tab = jnp.pad(bout.reshape(1, 1), ((0, 0), (0, D - 1)))

    degp = _sc_agg(ones_tab, packed32, zeros_tab)
    hs, dinv = _tc_first(x_pad, W0, degp)

    for b_prev, W in ((b0, W1), (b1, W2), (b2, W3)):
        aggp = _sc_agg(hs, packed32, zeros_tab)
        hs = _tc_mid(aggp, hs, dinv, b_prev.reshape(1, D), W)

    aggp = _sc_agg(hs, packed32, zeros_tab)
    out = _tc_last(aggp, hs, dinv, b3.reshape(1, D), wout_pad, bout_tab)
    return out[:N_NODES, :1]


# speedup vs baseline: 3.5545x; 2.2077x over previous
import functools

import jax
import jax.numpy as jnp
from jax import lax
from jax.experimental import pallas as pl
from jax.experimental.pallas import tpu as pltpu
from jax.experimental.pallas import tpu_sc as plsc

N_NODES = 10000
NPAD = 10240
D = 128
E = 320000
NC = 2
NS = 16
NW = NC * NS
EPW = E // NW
K = 80
ITERS = EPW // K
RPS = NPAD // NS


_sc_mesh = plsc.VectorSubcoreMesh(core_axis_name="c", subcore_axis_name="s")


@functools.partial(
    pl.kernel,
    mesh=_sc_mesh,
    out_type=jax.ShapeDtypeStruct((NC, NPAD, D), jnp.float32),
    scratch_types=[
        pltpu.VMEM((ITERS, K), jnp.int32),
        pltpu.VMEM((ITERS, K), jnp.int32),
        pltpu.VMEM((K, D), jnp.float32),
        pltpu.VMEM_SHARED((NPAD, D), jnp.float32),
        pltpu.SemaphoreType.DMA,
    ],
)
def _sc_agg(hs_hbm, srcr_hbm, dstr_hbm, zeros_hbm, out_hbm,
            src_v, dst_v, rows_v, agg_sh, sem):
    cid = lax.axis_index("c")
    sid = lax.axis_index("s")
    wid = sid * NC + cid

    pltpu.sync_copy(zeros_hbm.at[pl.ds(sid * RPS, RPS)],
                    agg_sh.at[pl.ds(sid * RPS, RPS)])
    pltpu.sync_copy(srcr_hbm.at[wid], src_v)
    pltpu.sync_copy(dstr_hbm.at[wid], dst_v)
    plsc.subcore_barrier()

    def body(i, carry):
        pltpu.async_copy(hs_hbm.at[src_v.at[i]], rows_v, sem).wait()
        pltpu.sync_copy(rows_v, agg_sh.at[dst_v.at[i]], add=True)
        return carry

    lax.fori_loop(0, ITERS, body, 0)
    plsc.subcore_barrier()
    pltpu.sync_copy(agg_sh.at[pl.ds(sid * RPS, RPS)],
                    out_hbm.at[cid, pl.ds(sid * RPS, RPS)])


_BR = 1024


def _tc_first_body(x_ref, w_ref, degp_ref, hs_ref, dinv_ref):
    deg = degp_ref[0, :, 0:1] + degp_ref[1, :, 0:1] + 1.0
    dinv = lax.rsqrt(deg)
    h = jnp.dot(x_ref[...], w_ref[...], preferred_element_type=jnp.float32)
    hs_ref[...] = h * dinv
    dinv_ref[...] = dinv


_tc_first = pl.pallas_call(
    _tc_first_body,
    grid=(NPAD // _BR,),
    in_specs=[
        pl.BlockSpec((_BR, D), lambda i: (i, 0)),
        pl.BlockSpec((D, D), lambda i: (0, 0)),
        pl.BlockSpec((NC, _BR, D), lambda i: (0, i, 0)),
    ],
    out_specs=[
        pl.BlockSpec((_BR, D), lambda i: (i, 0)),
        pl.BlockSpec((_BR, 1), lambda i: (i, 0)),
    ],
    out_shape=[
        jax.ShapeDtypeStruct((NPAD, D), jnp.float32),
        jax.ShapeDtypeStruct((NPAD, 1), jnp.float32),
    ],
)


def _tc_mid_body(aggp_ref, hs_ref, dinv_ref, b_ref, w_ref, out_ref):
    dinv = dinv_ref[...]
    pre = dinv * (aggp_ref[0] + aggp_ref[1] + hs_ref[...]) + b_ref[...]
    pre = jnp.maximum(pre, 0.0)
    h = jnp.dot(pre, w_ref[...], preferred_element_type=jnp.float32)
    out_ref[...] = h * dinv


def _tc_last_body(aggp_ref, hs_ref, dinv_ref, b_ref, w_ref, bout_ref, out_ref):
    dinv = dinv_ref[...]
    pre = dinv * (aggp_ref[0] + aggp_ref[1] + hs_ref[...]) + b_ref[...]
    pre = jnp.maximum(pre, 0.0)
    h = jnp.dot(pre, w_ref[...], preferred_element_type=jnp.float32)
    out_ref[...] = h + bout_ref[...]


def _tc_layer_call(body, n_extra):
    in_specs = [
        pl.BlockSpec((NC, _BR, D), lambda i: (0, i, 0)),
        pl.BlockSpec((_BR, D), lambda i: (i, 0)),
        pl.BlockSpec((_BR, 1), lambda i: (i, 0)),
        pl.BlockSpec((1, D), lambda i: (0, 0)),
        pl.BlockSpec((D, D), lambda i: (0, 0)),
    ]
    in_specs += [pl.BlockSpec((1, D), lambda i: (0, 0))] * n_extra
    return pl.pallas_call(
        body,
        grid=(NPAD // _BR,),
        in_specs=in_specs,
        out_specs=pl.BlockSpec((_BR, D), lambda i: (i, 0)),
        out_shape=jax.ShapeDtypeStruct((NPAD, D), jnp.float32),
    )


_tc_mid = _tc_layer_call(_tc_mid_body, 0)
_tc_last = _tc_layer_call(_tc_last_body, 1)


def kernel(x, edge_index, W0, b0, W1, b1, W2, b2, W3, b3, Wout, bout):
    src = edge_index[0].astype(jnp.int32).reshape(NW, ITERS, K)
    dst = edge_index[1].astype(jnp.int32).reshape(NW, ITERS, K)
    x_pad = jnp.pad(x, ((0, NPAD - N_NODES), (0, 0)))
    zeros_tab = jnp.zeros((NPAD, D), jnp.float32)
    ones_tab = jnp.ones((NPAD, D), jnp.float32)
    wout_pad = jnp.pad(Wout, ((0, 0), (0, D - Wout.shape[1])))
    bout_tab = jnp.pad(bout.reshape(1, 1), ((0, 0), (0, D - 1)))

    degp = _sc_agg(ones_tab, src, dst, zeros_tab)
    hs, dinv = _tc_first(x_pad, W0, degp)

    for b_prev, W in ((b0, W1), (b1, W2), (b2, W3)):
        aggp = _sc_agg(hs, src, dst, zeros_tab)
        hs = _tc_mid(aggp, hs, dinv, b_prev.reshape(1, D), W)

    aggp = _sc_agg(hs, src, dst, zeros_tab)
    out = _tc_last(aggp, hs, dinv, b3.reshape(1, D), wout_pad, bout_tab)
    return out[:N_NODES, :1]


# speedup vs baseline: 4.5352x; 1.2759x over previous
import functools

import jax
import jax.numpy as jnp
from jax import lax
from jax.experimental import pallas as pl
from jax.experimental.pallas import tpu as pltpu
from jax.experimental.pallas import tpu_sc as plsc

N_NODES = 10000
NPAD = 10240
D = 128
E = 320000
NC = 2
NS = 16
NW = NC * NS
EPW = E // NW
K = 80
ITERS = EPW // K
RPS = NPAD // NS


_sc_mesh = plsc.VectorSubcoreMesh(core_axis_name="c", subcore_axis_name="s")


@functools.partial(
    pl.kernel,
    mesh=_sc_mesh,
    out_type=jax.ShapeDtypeStruct((NC, NPAD, D), jnp.float32),
    scratch_types=[
        pltpu.VMEM((ITERS, K), jnp.int32),
        pltpu.VMEM((2, K), jnp.int32),
        pltpu.VMEM((2, K), jnp.int32),
        pltpu.VMEM((K, D), jnp.float32),
        pltpu.VMEM((K, D), jnp.float32),
        pltpu.VMEM_SHARED((NPAD, D), jnp.float32),
        pltpu.SemaphoreType.DMA((2,)),
    ],
)
def _sc_agg(hs_hbm, pk_hbm, zeros_hbm, out_hbm,
            pk_v, srcb, dstb, rows_a, rows_b, agg_sh, gsem):
    rows = (rows_a, rows_b)
    cid = lax.axis_index("c")
    sid = lax.axis_index("s")
    wid = sid * NC + cid

    pltpu.sync_copy(zeros_hbm.at[pl.ds(sid * RPS, RPS)],
                    agg_sh.at[pl.ds(sid * RPS, RPS)])
    pltpu.sync_copy(pk_hbm.at[wid], pk_v)
    plsc.subcore_barrier()

    def unpack(c, b):
        for j in range(K // 16):
            v = pk_v[c, pl.ds(j * 16, 16)]
            srcb[b, pl.ds(j * 16, 16)] = v & ((1 << 14) - 1)
            dstb[b, pl.ds(j * 16, 16)] = lax.shift_right_logical(v, 14)

    def gather(b):
        pltpu.async_copy(hs_hbm.at[srcb.at[b]], rows[b], gsem.at[b])

    def gwait(b):
        pltpu.make_async_copy(hs_hbm.at[pl.ds(0, K)], rows[b],
                              gsem.at[b]).wait()

    unpack(0, 0)
    gather(0)

    def body(go, carry):
        for j in range(2):
            c = go * 2 + j
            gwait(j)
            unpack(c + 1, 1 - j)
            gather(1 - j)
            pltpu.sync_copy(rows[j], agg_sh.at[dstb.at[j]], add=True)
        return carry

    lax.fori_loop(0, ITERS // 2, body, 0)
    gwait(0)
    pltpu.sync_copy(rows[0], agg_sh.at[dstb.at[0]], add=True)
    plsc.subcore_barrier()
    pltpu.sync_copy(agg_sh.at[pl.ds(sid * RPS, RPS)],
                    out_hbm.at[cid, pl.ds(sid * RPS, RPS)])


_BR = 1024


def _tc_first_body(x_ref, w_ref, degp_ref, hs_ref, dinv_ref):
    deg = degp_ref[0, :, 0:1] + degp_ref[1, :, 0:1] + 1.0
    dinv = lax.rsqrt(deg)
    h = jnp.dot(x_ref[...], w_ref[...], preferred_element_type=jnp.float32)
    hs_ref[...] = h * dinv
    dinv_ref[...] = dinv


_tc_first = pl.pallas_call(
    _tc_first_body,
    grid=(NPAD // _BR,),
    in_specs=[
        pl.BlockSpec((_BR, D), lambda i: (i, 0)),
        pl.BlockSpec((D, D), lambda i: (0, 0)),
        pl.BlockSpec((NC, _BR, D), lambda i: (0, i, 0)),
    ],
    out_specs=[
        pl.BlockSpec((_BR, D), lambda i: (i, 0)),
        pl.BlockSpec((_BR, 1), lambda i: (i, 0)),
    ],
    out_shape=[
        jax.ShapeDtypeStruct((NPAD, D), jnp.float32),
        jax.ShapeDtypeStruct((NPAD, 1), jnp.float32),
    ],
)


def _tc_mid_body(aggp_ref, hs_ref, dinv_ref, b_ref, w_ref, out_ref):
    dinv = dinv_ref[...]
    pre = dinv * (aggp_ref[0] + aggp_ref[1] + hs_ref[...]) + b_ref[...]
    pre = jnp.maximum(pre, 0.0)
    h = jnp.dot(pre, w_ref[...], preferred_element_type=jnp.float32)
    out_ref[...] = h * dinv


def _tc_last_body(aggp_ref, hs_ref, dinv_ref, b_ref, w_ref, bout_ref, out_ref):
    dinv = dinv_ref[...]
    pre = dinv * (aggp_ref[0] + aggp_ref[1] + hs_ref[...]) + b_ref[...]
    pre = jnp.maximum(pre, 0.0)
    h = jnp.dot(pre, w_ref[...], preferred_element_type=jnp.float32)
    out_ref[...] = h + bout_ref[...]


def _tc_layer_call(body, n_extra):
    in_specs = [
        pl.BlockSpec((NC, _BR, D), lambda i: (0, i, 0)),
        pl.BlockSpec((_BR, D), lambda i: (i, 0)),
        pl.BlockSpec((_BR, 1), lambda i: (i, 0)),
        pl.BlockSpec((1, D), lambda i: (0, 0)),
        pl.BlockSpec((D, D), lambda i: (0, 0)),
    ]
    in_specs += [pl.BlockSpec((1, D), lambda i: (0, 0))] * n_extra
    return pl.pallas_call(
        body,
        grid=(NPAD // _BR,),
        in_specs=in_specs,
        out_specs=pl.BlockSpec((_BR, D), lambda i: (i, 0)),
        out_shape=jax.ShapeDtypeStruct((NPAD, D), jnp.float32),
    )


_tc_mid = _tc_layer_call(_tc_mid_body, 0)
_tc_last = _tc_layer_call(_tc_last_body, 1)


def kernel(x, edge_index, W0, b0, W1, b1, W2, b2, W3, b3, Wout, bout):
    src = edge_index[0].astype(jnp.int32)
    dst = edge_index[1].astype(jnp.int32)
    packed = (src | (dst << 14)).reshape(NW, ITERS, K)
    x_pad = jnp.pad(x, ((0, NPAD - N_NODES), (0, 0)))
    zeros_tab = jnp.zeros((NPAD, D), jnp.float32)
    ones_tab = jnp.ones((NPAD, D), jnp.float32)
    wout_pad = jnp.pad(Wout, ((0, 0), (0, D - Wout.shape[1])))
    bout_tab = jnp.pad(bout.reshape(1, 1), ((0, 0), (0, D - 1)))

    degp = _sc_agg(ones_tab, packed, zeros_tab)
    hs, dinv = _tc_first(x_pad, W0, degp)

    for b_prev, W in ((b0, W1), (b1, W2), (b2, W3)):
        aggp = _sc_agg(hs, packed, zeros_tab)
        hs = _tc_mid(aggp, hs, dinv, b_prev.reshape(1, D), W)

    aggp = _sc_agg(hs, packed, zeros_tab)
    out = _tc_last(aggp, hs, dinv, b3.reshape(1, D), wout_pad, bout_tab)
    return out[:N_NODES, :1]
